# Initial kernel scaffold; baseline (speedup 1.0000x reference)
import jax
import jax.numpy as jnp
from jax.experimental import pallas as pl


def kernel(variate_ids, value_num, cat_ids, variate_type, numeric_means, numeric_stds, w1, b1, W2, b2, cat_table, ln_gamma, ln_beta):
    raise NotImplementedError("write your pallas kernel here")


# speedup vs baseline: 1.8452x; 1.8452x over previous
import functools
import jax
import jax.numpy as jnp
from jax.experimental import pallas as pl
from jax.experimental.pallas import tpu as pltpu

_B, _L, _D = 1024, 200, 128
_NV, _NCAT, _H = 64, 512, 16
_N = _B * _L
_T = 512


def _tc_body(vid_ref, cid_ref, val_ref, meta_ref, w1_ref, b1_ref, W2_ref,
             b2_ref, cat_ref, gam_ref, bet_ref, out_ref):
    vid = vid_ref[...]
    cid = cid_ref[...]
    val = val_ref[...]
    ohv = (vid == jax.lax.broadcasted_iota(jnp.int32, (_T, _NV), 1)
           ).astype(jnp.float32)
    g = jnp.dot(ohv, meta_ref[...], preferred_element_type=jnp.float32)
    mnum = g[:, 0:1]
    mu = g[:, 1:2]
    sg = g[:, 2:3]
    mcat = g[:, 3:4] * (cid >= 0).astype(jnp.float32)
    v = (val - mu) / (sg + 1e-6)
    h = jnp.maximum(v * w1_ref[...] + b1_ref[...], 0.0)
    e_num = jnp.dot(h, W2_ref[...], preferred_element_type=jnp.float32)
    e_num = e_num + b2_ref[...]
    ohc = (cid == jax.lax.broadcasted_iota(jnp.int32, (_T, _NCAT), 1)
           ).astype(jnp.float32)
    e_cat = jnp.dot(ohc, cat_ref[...], preferred_element_type=jnp.float32)
    e = mcat * e_cat + (1.0 - mcat) * (mnum * e_num)
    mean = jnp.mean(e, axis=1, keepdims=True)
    var = jnp.mean((e - mean) ** 2, axis=1, keepdims=True)
    out_ref[...] = (e - mean) / jnp.sqrt(var + 1e-5) * gam_ref[...] + bet_ref[...]


@jax.jit
def kernel(variate_ids, value_num, cat_ids, variate_type, numeric_means,
           numeric_stds, w1, b1, W2, b2, cat_table, ln_gamma, ln_beta):
    vid = variate_ids.reshape(_N, 1).astype(jnp.int32)
    cid = cat_ids.reshape(_N, 1).astype(jnp.int32)
    val = value_num.reshape(_N, 1)
    tf = variate_type.astype(jnp.int32)
    meta = jnp.zeros((_NV, _D), jnp.float32)
    meta = meta.at[:, 0].set((tf == 0).astype(jnp.float32))
    meta = meta.at[:, 1].set(numeric_means)
    meta = meta.at[:, 2].set(numeric_stds)
    meta = meta.at[:, 3].set((tf == 1).astype(jnp.float32))

    grid = _N // _T
    tok = lambda i: (i, 0)
    full = lambda i: (0, 0)
    out = pl.pallas_call(
        _tc_body,
        grid=(grid,),
        in_specs=[
            pl.BlockSpec((_T, 1), tok),
            pl.BlockSpec((_T, 1), tok),
            pl.BlockSpec((_T, 1), tok),
            pl.BlockSpec((_NV, _D), full),
            pl.BlockSpec((1, _H), full),
            pl.BlockSpec((1, _H), full),
            pl.BlockSpec((_H, _D), full),
            pl.BlockSpec((1, _D), full),
            pl.BlockSpec((_NCAT, _D), full),
            pl.BlockSpec((1, _D), full),
            pl.BlockSpec((1, _D), full),
        ],
        out_specs=pl.BlockSpec((_T, _D), tok),
        out_shape=jax.ShapeDtypeStruct((_N, _D), jnp.float32),
    )(vid, cid, val, meta, w1.reshape(1, _H), b1.reshape(1, _H), W2,
      b2.reshape(1, _D), cat_table, ln_gamma.reshape(1, _D),
      ln_beta.reshape(1, _D))
    return out.reshape(_B, _L, _D)
